# baseline (device time: 34273 ns/iter reference)
import jax
import jax.numpy as jnp
from jax import lax
from jax.experimental import pallas as pl
from jax.experimental.pallas import tpu as pltpu

N_DEV = 8
M = 1024
N = 1024
N_STAGES = 3
N_STEPS = 7

PARTS = [
    (0, 384, (0, 1, 2)),
    (384, 384, (1, 2, 0)),
    (768, 256, (2, 0, 1)),
]


def _coords(my):
    q = my % 4
    cx = jnp.where((q == 1) | (q == 2), 1, 0).astype(jnp.int32)
    cy = q // 2
    cz = my // 4
    return (cx, cy, cz)


def _partner(my, axis):
    q = my % 4
    if axis == 0:
        return my + 1 - 2 * (q % 2)
    if axis == 1:
        return my - 2 * q + 3
    return (my + 4) % N_DEV


def kernel(x, w_mat):
    n_parts = len(PARTS)

    def body(x_ref, w_ref, out_ref, *scratch):
        bufs = scratch[: 7 * n_parts]
        send_sems, recv_sems = scratch[7 * n_parts:]

        def sb(p, s):
            return bufs[7 * p + 2 * s]

        def rb(p, s):
            return bufs[7 * p + 2 * s + 1]

        def shadow(p):
            return bufs[7 * p + 6]

        my = lax.axis_index("i")
        coords = _coords(my)
        all_rdmas = []

        def exchange(p, step, src, dst, axis):
            rdma = pltpu.make_async_remote_copy(
                src_ref=src,
                dst_ref=dst,
                send_sem=send_sems.at[N_STEPS * p + step],
                recv_sem=recv_sems.at[N_STEPS * p + step],
                device_id=_partner(my, axis),
                device_id_type=pl.DeviceIdType.LOGICAL,
            )
            all_rdmas.append(rdma)
            rdma.start()
            return rdma

        barrier = pltpu.get_barrier_semaphore()
        for axis in range(3):
            pl.semaphore_signal(
                barrier,
                inc=1,
                device_id=_partner(my, axis),
                device_id_type=pl.DeviceIdType.LOGICAL,
            )
        pl.semaphore_wait(barrier, 3)

        out_ref[:, :] = jnp.dot(
            x_ref[:, :].astype(jnp.bfloat16),
            w_ref[:, :].astype(jnp.bfloat16),
            preferred_element_type=jnp.float32,
        )

        bases = [jnp.int32(0) for _ in PARTS]
        rdmas = []
        for p, (c0, nc, order) in enumerate(PARTS):
            c = coords[order[0]]
            send = bases[p] + (1 - c) * (M // 2)
            sb(p, 0)[:, :] = out_ref[
                pl.ds(send, M // 2), pl.ds(c0, nc)
            ].astype(jnp.bfloat16)
            rdmas.append(exchange(p, 0, sb(p, 0), rb(p, 0), order[0]))

        for s in range(N_STAGES - 1):
            L = M >> (s + 1)
            L2 = L // 2
            next_rdmas = []
            rels = []
            for p, (c0, nc, order) in enumerate(PARTS):
                c = coords[order[s]]
                keep = bases[p] + c * L
                cn = coords[order[s + 1]]
                rel_send = (1 - cn) * L2
                rel_keep = cn * L2
                rdmas[p].wait_recv()
                cols = pl.ds(c0, nc)
                sb(p, s + 1)[:, :] = (
                    out_ref[pl.ds(keep + rel_send, L2), cols]
                    + rb(p, s)[pl.ds(rel_send, L2), :].astype(jnp.float32)
                ).astype(jnp.bfloat16)
                next_rdmas.append(
                    exchange(p, s + 1, sb(p, s + 1), rb(p, s + 1), order[s + 1])
                )
                bases[p] = keep
                rels.append((rel_send, rel_keep))
            for p, (c0, nc, order) in enumerate(PARTS):
                rel_send, rel_keep = rels[p]
                cols = pl.ds(c0, nc)
                for rel in (rel_send, rel_keep):
                    out_ref[pl.ds(bases[p] + rel, L2), cols] = out_ref[
                        pl.ds(bases[p] + rel, L2), cols
                    ] + rb(p, s)[pl.ds(rel, L2), :].astype(jnp.float32)
            rdmas = next_rdmas

        Lf = M >> N_STAGES
        next_rdmas = []
        for p, (c0, nc, order) in enumerate(PARTS):
            c = coords[order[N_STAGES - 1]]
            keep = bases[p] + c * Lf
            rdmas[p].wait_recv()
            shadow(p)[pl.ds(keep, Lf), :] = (
                out_ref[pl.ds(keep, Lf), pl.ds(c0, nc)]
                + rb(p, 2)[:, :].astype(jnp.float32)
            ).astype(jnp.bfloat16)
            next_rdmas.append(
                exchange(
                    p,
                    3,
                    shadow(p).at[pl.ds(keep, Lf), :],
                    shadow(p).at[pl.ds(keep, Lf), :],
                    order[N_STAGES - 1],
                )
            )
            bases[p] = keep
        for p, (c0, nc, order) in enumerate(PARTS):
            out_ref[pl.ds(bases[p], Lf), pl.ds(c0, nc)] = out_ref[
                pl.ds(bases[p], Lf), pl.ds(c0, nc)
            ] + rb(p, 2)[:, :].astype(jnp.float32)
        rdmas = next_rdmas

        pbases = []
        for t in range(N_STAGES - 1):
            L = M >> (N_STAGES - t)
            next_rdmas = []
            pbases = []
            for p, (c0, nc, order) in enumerate(PARTS):
                c = coords[order[N_STAGES - 1 - t]]
                parent = bases[p] - c * L
                pbases.append(parent + (1 - c) * L)
                rdmas[p].wait_recv()
                if t < N_STAGES - 2:
                    next_rdmas.append(
                        exchange(
                            p,
                            4,
                            shadow(p).at[pl.ds(parent, 2 * L), :],
                            shadow(p).at[pl.ds(parent, 2 * L), :],
                            order[N_STAGES - 2 - t],
                        )
                    )
                else:
                    halves = []
                    for h in range(2):
                        halves.append(
                            exchange(
                                p,
                                5 + h,
                                shadow(p).at[pl.ds(parent + h * L, L), :],
                                shadow(p).at[pl.ds(parent + h * L, L), :],
                                order[0],
                            )
                        )
                    next_rdmas.append(halves)
                bases[p] = parent
            for p, (c0, nc, order) in enumerate(PARTS):
                out_ref[pl.ds(pbases[p], L), pl.ds(c0, nc)] = shadow(p)[
                    pl.ds(pbases[p], L), :
                ].astype(jnp.float32)
            rdmas = next_rdmas

        Lh = M // 4
        pbases = []
        for p, (c0, nc, order) in enumerate(PARTS):
            c = coords[order[0]]
            parent = bases[p] - c * (M // 2)
            pbases.append(parent + (1 - c) * (M // 2))
            bases[p] = parent
        for h in range(2):
            for p, (c0, nc, order) in enumerate(PARTS):
                rdmas[p][h].wait_recv()
            for p, (c0, nc, order) in enumerate(PARTS):
                pb = pbases[p] + h * Lh
                out_ref[pl.ds(pb, Lh), pl.ds(c0, nc)] = shadow(p)[
                    pl.ds(pb, Lh), :
                ].astype(jnp.float32)

        for r in all_rdmas:
            r.wait_send()

    scratch_shapes = []
    for (c0, nc, order) in PARTS:
        for s in range(N_STAGES):
            L = M >> (s + 1)
            scratch_shapes += [pltpu.VMEM((L, nc), jnp.bfloat16)] * 2
        scratch_shapes.append(pltpu.VMEM((M, nc), jnp.bfloat16))
    n_sems = n_parts * N_STEPS
    scratch_shapes += [
        pltpu.SemaphoreType.DMA((n_sems,)),
        pltpu.SemaphoreType.DMA((n_sems,)),
    ]

    return pl.pallas_call(
        body,
        out_shape=jax.ShapeDtypeStruct((M, N), jnp.float32),
        in_specs=[
            pl.BlockSpec(memory_space=pltpu.VMEM),
            pl.BlockSpec(memory_space=pltpu.VMEM),
        ],
        out_specs=pl.BlockSpec(memory_space=pltpu.VMEM),
        scratch_shapes=scratch_shapes,
        compiler_params=pltpu.CompilerParams(collective_id=0),
    )(x, w_mat)


# device time: 27631 ns/iter; 1.2404x vs baseline; 1.2404x over previous
import jax
import jax.numpy as jnp
from jax import lax
from jax.experimental import pallas as pl
from jax.experimental.pallas import tpu as pltpu

N_DEV = 8
M = 1024
N = 1024
H = M // 4
N_WINDOWS = 4
N_STEPS = 2 * N_WINDOWS

PARTS = [
    (0, 384, (0, 1, 2)),
    (384, 384, (1, 2, 0)),
    (768, 256, (2, 0, 1)),
]


def _coords(my):
    q = my % 4
    cx = jnp.where((q == 1) | (q == 2), 1, 0).astype(jnp.int32)
    cy = q // 2
    cz = my // 4
    return (cx, cy, cz)


def _partner(my, axis):
    q = my % 4
    if axis == 0:
        return my + 1 - 2 * (q % 2)
    if axis == 1:
        return my - 2 * q + 3
    return (my + 4) % N_DEV


def kernel(x, w_mat):
    n_parts = len(PARTS)

    def body(x_ref, w_ref, out_ref, *scratch):
        bufs = scratch[: N_STEPS * n_parts]
        send_sems, recv_sems = scratch[N_STEPS * n_parts:]

        def sb(p, w):
            return bufs[N_STEPS * p + 2 * w]

        def rb(p, w):
            return bufs[N_STEPS * p + 2 * w + 1]

        my = lax.axis_index("i")
        coords = _coords(my)
        all_rdmas = []

        def exchange(p, w, h, src, dst, axis):
            rdma = pltpu.make_async_remote_copy(
                src_ref=src,
                dst_ref=dst,
                send_sem=send_sems.at[N_STEPS * p + 2 * w + h],
                recv_sem=recv_sems.at[N_STEPS * p + 2 * w + h],
                device_id=_partner(my, axis),
                device_id_type=pl.DeviceIdType.LOGICAL,
            )
            all_rdmas.append(rdma)
            rdma.start()
            return rdma

        barrier = pltpu.get_barrier_semaphore()
        for axis in range(3):
            pl.semaphore_signal(
                barrier,
                inc=1,
                device_id=_partner(my, axis),
                device_id_type=pl.DeviceIdType.LOGICAL,
            )
        pl.semaphore_wait(barrier, 3)

        out_ref[:, :] = jnp.dot(
            x_ref[:, :].astype(jnp.bfloat16),
            w_ref[:, :].astype(jnp.bfloat16),
            preferred_element_type=jnp.float32,
        )

        keep0 = []
        send0 = []
        for p, (c0, nc, order) in enumerate(PARTS):
            c = coords[order[0]]
            keep0.append(c * (M // 2))
            send0.append((1 - c) * (M // 2))

        rdmas = {}

        for h in range(2):
            for p, (c0, nc, order) in enumerate(PARTS):
                sb(p, 0)[pl.ds(h * H, H), :] = out_ref[
                    pl.ds(send0[p] + h * H, H), pl.ds(c0, nc)
                ].astype(jnp.bfloat16)
                rdmas[(p, 0, h)] = exchange(
                    p,
                    0,
                    h,
                    sb(p, 0).at[pl.ds(h * H, H), :],
                    rb(p, 0).at[pl.ds(h * H, H), :],
                    order[0],
                )

        for w in range(3):
            for h in range(2):
                for p, (c0, nc, order) in enumerate(PARTS):
                    rows = pl.ds(keep0[p] + h * H, H)
                    rel = pl.ds(h * H, H)
                    rdmas[(p, w, h)].wait_recv()
                    val = out_ref[rows, pl.ds(c0, nc)] + rb(p, w)[
                        rel, :
                    ].astype(jnp.float32)
                    sb(p, w + 1)[rel, :] = val.astype(jnp.bfloat16)
                    rdmas[(p, w + 1, h)] = exchange(
                        p,
                        w + 1,
                        h,
                        sb(p, w + 1).at[rel, :],
                        rb(p, w + 1).at[rel, :],
                        order[(w + 1) % 3],
                    )
                    out_ref[rows, pl.ds(c0, nc)] = val

        for h in range(2):
            for p, (c0, nc, order) in enumerate(PARTS):
                rdmas[(p, 3, h)].wait_recv()
                out_ref[pl.ds(send0[p] + h * H, H), pl.ds(c0, nc)] = rb(p, 3)[
                    pl.ds(h * H, H), :
                ].astype(jnp.float32)

        for r in all_rdmas:
            r.wait_send()

    scratch_shapes = []
    for (c0, nc, order) in PARTS:
        scratch_shapes += [pltpu.VMEM((M // 2, nc), jnp.bfloat16)] * N_STEPS
    n_sems = n_parts * N_STEPS
    scratch_shapes += [
        pltpu.SemaphoreType.DMA((n_sems,)),
        pltpu.SemaphoreType.DMA((n_sems,)),
    ]

    return pl.pallas_call(
        body,
        out_shape=jax.ShapeDtypeStruct((M, N), jnp.float32),
        in_specs=[
            pl.BlockSpec(memory_space=pltpu.VMEM),
            pl.BlockSpec(memory_space=pltpu.VMEM),
        ],
        out_specs=pl.BlockSpec(memory_space=pltpu.VMEM),
        scratch_shapes=scratch_shapes,
        compiler_params=pltpu.CompilerParams(collective_id=0),
    )(x, w_mat)


# device time: 25655 ns/iter; 1.3359x vs baseline; 1.0770x over previous
import jax
import jax.numpy as jnp
from jax import lax
from jax.experimental import pallas as pl
from jax.experimental.pallas import tpu as pltpu

N_DEV = 8
M = 1024
N = 1024
K = 4
H = M // 2 // K
N_WINDOWS = 4
N_STEPS = K * N_WINDOWS

PARTS = [
    (0, 384, (0, 1, 2)),
    (384, 384, (1, 2, 0)),
    (768, 256, (2, 0, 1)),
]


def _coords(my):
    q = my % 4
    cx = jnp.where((q == 1) | (q == 2), 1, 0).astype(jnp.int32)
    cy = q // 2
    cz = my // 4
    return (cx, cy, cz)


def _partner(my, axis):
    q = my % 4
    if axis == 0:
        return my + 1 - 2 * (q % 2)
    if axis == 1:
        return my - 2 * q + 3
    return (my + 4) % N_DEV


def kernel(x, w_mat):
    n_parts = len(PARTS)

    def body(x_ref, w_ref, out_ref, *scratch):
        bufs = scratch[: 2 * N_WINDOWS * n_parts]
        send_sems, recv_sems = scratch[2 * N_WINDOWS * n_parts:]

        def sb(p, w):
            return bufs[2 * N_WINDOWS * p + 2 * w]

        def rb(p, w):
            return bufs[2 * N_WINDOWS * p + 2 * w + 1]

        my = lax.axis_index("i")
        coords = _coords(my)
        all_rdmas = []

        def exchange(p, w, h, src, dst, axis):
            rdma = pltpu.make_async_remote_copy(
                src_ref=src,
                dst_ref=dst,
                send_sem=send_sems.at[N_STEPS * p + K * w + h],
                recv_sem=recv_sems.at[N_STEPS * p + K * w + h],
                device_id=_partner(my, axis),
                device_id_type=pl.DeviceIdType.LOGICAL,
            )
            all_rdmas.append(rdma)
            rdma.start()
            return rdma

        barrier = pltpu.get_barrier_semaphore()
        for axis in range(3):
            pl.semaphore_signal(
                barrier,
                inc=1,
                device_id=_partner(my, axis),
                device_id_type=pl.DeviceIdType.LOGICAL,
            )
        pl.semaphore_wait(barrier, 3)

        out_ref[:, :] = jnp.dot(
            x_ref[:, :].astype(jnp.bfloat16),
            w_ref[:, :].astype(jnp.bfloat16),
            preferred_element_type=jnp.float32,
        )

        keep0 = []
        send0 = []
        for p, (c0, nc, order) in enumerate(PARTS):
            c = coords[order[0]]
            keep0.append(c * (M // 2))
            send0.append((1 - c) * (M // 2))

        rdmas = {}

        for h in range(K):
            for p, (c0, nc, order) in enumerate(PARTS):
                sb(p, 0)[pl.ds(h * H, H), :] = out_ref[
                    pl.ds(send0[p] + h * H, H), pl.ds(c0, nc)
                ].astype(jnp.bfloat16)
                rdmas[(p, 0, h)] = exchange(
                    p,
                    0,
                    h,
                    sb(p, 0).at[pl.ds(h * H, H), :],
                    rb(p, 0).at[pl.ds(h * H, H), :],
                    order[0],
                )

        for w in range(3):
            for h in range(K):
                for p, (c0, nc, order) in enumerate(PARTS):
                    rows = pl.ds(keep0[p] + h * H, H)
                    rel = pl.ds(h * H, H)
                    rdmas[(p, w, h)].wait_recv()
                    val = out_ref[rows, pl.ds(c0, nc)] + rb(p, w)[
                        rel, :
                    ].astype(jnp.float32)
                    sb(p, w + 1)[rel, :] = val.astype(jnp.bfloat16)
                    rdmas[(p, w + 1, h)] = exchange(
                        p,
                        w + 1,
                        h,
                        sb(p, w + 1).at[rel, :],
                        rb(p, w + 1).at[rel, :],
                        order[(w + 1) % 3],
                    )
                    out_ref[rows, pl.ds(c0, nc)] = val

        for h in range(K):
            for p, (c0, nc, order) in enumerate(PARTS):
                rdmas[(p, 3, h)].wait_recv()
                out_ref[pl.ds(send0[p] + h * H, H), pl.ds(c0, nc)] = rb(p, 3)[
                    pl.ds(h * H, H), :
                ].astype(jnp.float32)

        for r in all_rdmas:
            r.wait_send()

    scratch_shapes = []
    for (c0, nc, order) in PARTS:
        scratch_shapes += [pltpu.VMEM((M // 2, nc), jnp.bfloat16)] * (2 * N_WINDOWS)
    n_sems = n_parts * N_STEPS
    scratch_shapes += [
        pltpu.SemaphoreType.DMA((n_sems,)),
        pltpu.SemaphoreType.DMA((n_sems,)),
    ]

    return pl.pallas_call(
        body,
        out_shape=jax.ShapeDtypeStruct((M, N), jnp.float32),
        in_specs=[
            pl.BlockSpec(memory_space=pltpu.VMEM),
            pl.BlockSpec(memory_space=pltpu.VMEM),
        ],
        out_specs=pl.BlockSpec(memory_space=pltpu.VMEM),
        scratch_shapes=scratch_shapes,
        compiler_params=pltpu.CompilerParams(collective_id=0),
    )(x, w_mat)
